# Initial kernel scaffold; baseline (speedup 1.0000x reference)
import jax
import jax.numpy as jnp
from jax.experimental import pallas as pl


def kernel(x, edge_index, batch, W1, b1, W2, b2, W3, b3, pw1, pw2, pw3, L1W, L1b, L2W, L2b, L3W, L3b):
    raise NotImplementedError("write your pallas kernel here")


# speedup vs baseline: 1.0000x; 1.0000x over previous
import functools

import jax
import jax.numpy as jnp
from jax.experimental import pallas as pl
from jax.experimental.pallas import tpu as pltpu

N = 10000
E = 320000
F = 128
H = 128
C = 30
G = 64
RATIO = 0.5


def _leaky(v):
    return jnp.where(v > 0, v, 0.1 * v)


def _safe_rsqrt(d):
    return jnp.where(d > 0, 1.0 / jnp.sqrt(jnp.where(d > 0, d, 1.0)), 0.0)


def _gcn_conv(x, src, dst, keep, emask, W, b):
    n = x.shape[0]
    loop = jnp.arange(n)
    s2 = jnp.concatenate([src, loop])
    d2 = jnp.concatenate([dst, loop])
    ew = jnp.concatenate([emask.astype(x.dtype), keep.astype(x.dtype)])
    deg = jnp.zeros((n,), x.dtype).at[d2].add(ew)
    dis = _safe_rsqrt(deg)
    norm = dis[s2] * ew * dis[d2]
    h = x @ W
    out = jnp.zeros((n, W.shape[1]), x.dtype).at[d2].add(norm[:, None] * h[s2])
    return out + b


def _topk_pool(x, src, dst, batch, keep, emask, w):
    n = x.shape[0]
    score = jnp.tanh((x @ w) / jnp.linalg.norm(w))
    valid_counts = jnp.bincount(batch, weights=keep.astype(x.dtype), length=G)
    k = jnp.ceil(RATIO * valid_counts).astype(jnp.int32)
    key_score = jnp.where(keep, score, -1e9)
    order = jnp.lexsort((-key_score, batch))
    counts_all = jnp.bincount(batch, length=G)
    starts = jnp.concatenate([jnp.zeros((1,), counts_all.dtype), jnp.cumsum(counts_all)[:-1]])
    rank = jnp.arange(n) - starts[batch[order]]
    keep_sorted = rank < k[batch[order]]
    new_keep = jnp.zeros((n,), bool).at[order].set(keep_sorted) & keep
    x_new = jnp.where(new_keep[:, None], x * score[:, None], 0.0)
    emask_new = emask & new_keep[src] & new_keep[dst]
    return x_new, new_keep, emask_new


def _gmax(x, batch, keep):
    xm = jnp.where(keep[:, None], x, -1e30)
    out = jnp.full((G, x.shape[1]), -1e30, x.dtype).at[batch].max(xm)
    cnt = jnp.bincount(batch, weights=keep.astype(x.dtype), length=G)
    return jnp.where((cnt > 0)[:, None], out, 0.0)


def _gmean(x, batch, keep):
    xs = jnp.where(keep[:, None], x, 0.0)
    s = jnp.zeros((G, x.shape[1]), x.dtype).at[batch].add(xs)
    cnt = jnp.bincount(batch, weights=keep.astype(x.dtype), length=G)
    return s / jnp.maximum(cnt, 1.0)[:, None]


def _node_info(x, src, dst, keep, emask):
    n = x.shape[0]
    ew = ((src != dst) & emask).astype(x.dtype)
    deg = jnp.zeros((n,), x.dtype).at[src].add(ew)
    dis = _safe_rsqrt(deg)
    norm = -(dis[src] * ew * dis[dst])
    out = jnp.zeros_like(x).at[dst].add(norm[:, None] * x[src])
    return out + x * keep.astype(x.dtype)[:, None]


def _mlp_kernel(x1_ref, x2_ref, x3_ref, w1_ref, b1_ref, w2_ref, b2_ref,
                w3_ref, b3_ref, out_ref):
    g = _leaky(x1_ref[...]) + _leaky(x2_ref[...]) + _leaky(x3_ref[...])
    g = _leaky(jnp.dot(g, w1_ref[...], preferred_element_type=jnp.float32) + b1_ref[...])
    g = _leaky(jnp.dot(g, w2_ref[...], preferred_element_type=jnp.float32) + b2_ref[...])
    out_ref[...] = jnp.dot(g, w3_ref[...], preferred_element_type=jnp.float32) + b3_ref[...]


def _mlp(x1, x2, x3, L1W, L1b, L2W, L2b, L3W, L3b):
    return pl.pallas_call(
        _mlp_kernel,
        out_shape=jax.ShapeDtypeStruct((G, C), jnp.float32),
    )(x1, x2, x3, L1W, L1b.reshape(1, H), L2W, L2b.reshape(1, H // 2),
      L3W, L3b.reshape(1, C))


def kernel(x, edge_index, batch, W1, b1, W2, b2, W3, b3, pw1, pw2, pw3,
           L1W, L1b, L2W, L2b, L3W, L3b):
    src = edge_index[0]
    dst = edge_index[1]
    n = x.shape[0]
    keep = jnp.ones((n,), bool)
    emask = jnp.ones((src.shape[0],), bool)
    h = _leaky(_gcn_conv(x, src, dst, keep, emask, W1, b1))
    h, keep, emask = _topk_pool(h, src, dst, batch, keep, emask, pw1)
    x1 = jnp.concatenate([_gmax(h, batch, keep), _gmean(h, batch, keep)], axis=1)
    h = _leaky(_gcn_conv(h, src, dst, keep, emask, W2, b2))
    h, keep, emask = _topk_pool(h, src, dst, batch, keep, emask, pw2)
    x2 = jnp.concatenate([_gmax(h, batch, keep), _gmean(h, batch, keep)], axis=1)
    h = _leaky(_gcn_conv(h, src, dst, keep, emask, W3, b3))
    h, keep, emask = _topk_pool(h, src, dst, batch, keep, emask, pw3)
    info = _node_info(h, src, dst, keep, emask)
    node_score = jnp.sum(jnp.abs(info), axis=1) * keep.astype(x.dtype)
    score_mean = jnp.sum(node_score) / jnp.maximum(jnp.sum(keep.astype(x.dtype)), 1.0)
    x3 = jnp.concatenate([_gmax(h, batch, keep), _gmean(h, batch, keep)], axis=1)
    logits = _mlp(x1, x2, x3, L1W, L1b, L2W, L2b, L3W, L3b)
    return logits, score_mean


# speedup vs baseline: 18.6338x; 18.6335x over previous
import functools

import numpy as np
import jax
import jax.numpy as jnp
from jax import lax
from jax.experimental import pallas as pl
from jax.experimental.pallas import tpu as pltpu
from jax.experimental.pallas import tpu_sc as plsc

N = 10000
E = 320000
F = 128
H = 128
C = 30
G = 64

NC = 2
NS = 16
NW = NC * NS
LANES = 16

NP = 10240
DUMMY = N
NBLK = 79
EW = NBLK * 128
EP = NW * EW
ROWS_PER_TILE = NP // NS


def _map_const(v):
    ui = np.array(v, np.float32).view(np.int32)
    return int(ui ^ ((ui >> np.int32(31)) & np.int32(0x7FFFFFFF)))


_MAP_LO = _map_const(-1e9)
_MAP_HI = _map_const(1.0) + 1


def _leaky(v):
    return jnp.where(v > 0, v, 0.1 * v)


def _safe_rsqrt(d):
    return jnp.where(d > 0, 1.0 / jnp.sqrt(jnp.where(d > 0, d, 1.0)), 0.0)


_SC_MESH = plsc.VectorSubcoreMesh(core_axis_name="c", subcore_axis_name="s")


def _make_edge_prep(with_keep):
    scratch = [
        pltpu.VMEM((NBLK, 128), jnp.int32),
        pltpu.VMEM((NBLK, 128), jnp.int32),
        pltpu.VMEM((NP,), jnp.float32),
    ]
    if with_keep:
        scratch.insert(0, pltpu.VMEM((NP,), jnp.float32))

    def body(*refs):
        if with_keep:
            (keep_hbm, src_hbm, dstp_hbm, dstp_out, deg_out,
             keepv, srcb, dstpb, degv) = refs
        else:
            (src_hbm, dstp_hbm, dstp_out, deg_out,
             srcb, dstpb, degv) = refs
        cid = lax.axis_index("c")
        sid = lax.axis_index("s")
        wid = sid * NC + cid
        if with_keep:
            pltpu.sync_copy(keep_hbm, keepv)
        pltpu.sync_copy(src_hbm.at[wid], srcb)
        pltpu.sync_copy(dstp_hbm.at[wid], dstpb)

        zero16 = jnp.zeros((LANES,), jnp.float32)

        def zbody(i, carry):
            degv[pl.ds(i * LANES, LANES)] = zero16
            return carry

        lax.fori_loop(0, NP // LANES, zbody, 0)

        one16 = jnp.ones((LANES,), jnp.float32)
        dummy16 = jnp.full((LANES,), DUMMY, jnp.int32)

        def ebody(j, carry):
            for i in range(128 // LANES):
                s16 = srcb[j, pl.ds(i * LANES, LANES)]
                dp16 = dstpb[j, pl.ds(i * LANES, LANES)]
                if with_keep:
                    ks = plsc.load_gather(keepv, [s16])
                    kd = plsc.load_gather(keepv, [dp16])
                    valid = (dp16 < N) & (ks > 0.0) & (kd > 0.0)
                    out16 = jnp.where(valid, dp16, dummy16)
                    degidx = out16
                else:
                    valid = (dp16 < N) & (s16 != dp16)
                    out16 = jnp.where(valid, dp16, dummy16)
                    degidx = jnp.where(valid, s16, dummy16)
                dstpb[j, pl.ds(i * LANES, LANES)] = out16
                plsc.addupdate_scatter(degv, [degidx], one16)
            return carry

        lax.fori_loop(0, NBLK, ebody, 0)

        pltpu.sync_copy(dstpb, dstp_out.at[wid])
        pltpu.sync_copy(degv, deg_out.at[wid])

    return pl.kernel(
        body,
        out_type=(
            jax.ShapeDtypeStruct((NW, NBLK, 128), jnp.int32),
            jax.ShapeDtypeStruct((NW, NP), jnp.float32),
        ),
        mesh=_SC_MESH,
        scratch_types=scratch,
        compiler_params=pltpu.CompilerParams(needs_layout_passes=False),
    )


_edge_prep_conv = _make_edge_prep(True)
_edge_prep_ni = _make_edge_prep(False)


def _propagate_body(hh_hbm, src_hbm, dstp_hbm, acc_out,
                    srcb, dstb, rows, acc, sem):
    cid = lax.axis_index("c")
    sid = lax.axis_index("s")
    wid = sid * NC + cid
    pltpu.sync_copy(src_hbm.at[wid], srcb)
    pltpu.sync_copy(dstp_hbm.at[wid], dstb)

    zero16 = jnp.zeros((LANES,), jnp.float32)

    def zrow(r, carry):
        for i in range(128 // LANES):
            rows[r, pl.ds(i * LANES, LANES)] = zero16
        return carry

    lax.fori_loop(0, 128, zrow, 0)
    base = sid * ROWS_PER_TILE
    for b in range(ROWS_PER_TILE // 128):
        pltpu.sync_copy(rows, acc.at[pl.ds(base + b * 128, 128)])
    plsc.subcore_barrier()

    def ebody(j, carry):
        pltpu.async_copy(hh_hbm.at[srcb.at[j]], rows, sem).wait()
        pltpu.sync_copy(rows, acc.at[dstb.at[j]], add=True)
        return carry

    lax.fori_loop(0, NBLK, ebody, 0)
    plsc.subcore_barrier()
    for b in range(ROWS_PER_TILE // 128):
        r0 = base + b * 128
        pltpu.sync_copy(acc.at[pl.ds(r0, 128)], acc_out.at[cid, pl.ds(r0, 128)])


_propagate = pl.kernel(
    _propagate_body,
    out_type=jax.ShapeDtypeStruct((NC, NP, 128), jnp.float32),
    mesh=_SC_MESH,
    scratch_types=[
        pltpu.VMEM((NBLK, 128), jnp.int32),
        pltpu.VMEM((NBLK, 128), jnp.int32),
        pltpu.VMEM((128, 128), jnp.float32),
        pltpu.VMEM_SHARED((NP, 128), jnp.float32),
        pltpu.SemaphoreType.DMA,
    ],
    compiler_params=pltpu.CompilerParams(needs_layout_passes=False),
)


def _degsum_body(p_ref, o_ref):
    o_ref[...] = jnp.sum(p_ref[...], axis=0)


def _tck_degsum(degp):
    return pl.pallas_call(
        _degsum_body,
        out_shape=jax.ShapeDtypeStruct((NP // 128, 128), jnp.float32),
    )(degp.reshape(NW, NP // 128, 128))


def _first_body(x_ref, w_ref, deg_ref, keep_ref, xw_ref, hh_ref):
    xw = jnp.dot(x_ref[...], w_ref[...], preferred_element_type=jnp.float32)
    deg = deg_ref[...] + keep_ref[...]
    dis = _safe_rsqrt(deg)
    xw_ref[...] = xw
    hh_ref[...] = dis * xw


def _tck_first(x, w, degp, keep):
    return pl.pallas_call(
        _first_body,
        out_shape=(
            jax.ShapeDtypeStruct((NP, H), jnp.float32),
            jax.ShapeDtypeStruct((NP, H), jnp.float32),
        ),
    )(x, w, degp, keep)


def _scale_body(xw_ref, deg_ref, keep_ref, hh_ref):
    deg = deg_ref[...] + keep_ref[...]
    hh_ref[...] = _safe_rsqrt(deg) * xw_ref[...]


def _tck_scale(xw, degp, keep):
    return pl.pallas_call(
        _scale_body,
        out_shape=jax.ShapeDtypeStruct((NP, H), jnp.float32),
    )(xw, degp, keep)


def _scale_ni_body(h_ref, deg_ref, hh_ref):
    deg = deg_ref[...]
    hh_ref[...] = _safe_rsqrt(deg) * h_ref[...]


def _tck_scale_ni(h, degp):
    return pl.pallas_call(
        _scale_ni_body,
        out_shape=jax.ShapeDtypeStruct((NP, H), jnp.float32),
    )(h, degp)


def _convscore_body(accp_ref, deg_ref, keep_ref, xw_ref, pw_ref, b_ref,
                    h_ref, score_ref):
    acc = accp_ref[0] + accp_ref[1]
    keep = keep_ref[...]
    deg = deg_ref[...] + keep
    dis = _safe_rsqrt(deg)
    h = _leaky(dis * acc + dis * dis * keep * xw_ref[...] + b_ref[...])
    pw = pw_ref[...]
    wnorm = jnp.sqrt(jnp.sum(pw * pw))
    h_ref[...] = h
    score_ref[...] = jnp.tanh(
        jnp.dot(h, pw, preferred_element_type=jnp.float32) / wnorm)


def _tck_convscore(accp, deg, keep, xw, pw, b):
    return pl.pallas_call(
        _convscore_body,
        out_shape=(
            jax.ShapeDtypeStruct((NP, H), jnp.float32),
            jax.ShapeDtypeStruct((NP, 1), jnp.float32),
        ),
    )(accp, deg, keep, xw, pw, b)


def _select_body(score_ref, keep_ref, batch_ref, keepn_ref):
    keep = keep_ref[...]
    keepb = keep > 0.0
    batch2 = batch_ref[...]
    gids = lax.broadcasted_iota(jnp.int32, (1, G), 1)
    oh = batch2 == gids
    ohf = oh.astype(jnp.float32)
    key = jnp.where(keepb, score_ref[...], jnp.float32(-1e9))
    ui = lax.bitcast_convert_type(key, jnp.int32)
    m = ui ^ ((ui >> 31) & jnp.int32(0x7FFFFFFF))

    vc = jnp.sum(ohf * keep, axis=0, keepdims=True)
    k = jnp.ceil(0.5 * vc)

    def bs(i, carry):
        lo, hi = carry
        mid = lo + lax.shift_right_logical(hi - lo, 1)
        ge = (m >= mid) & oh
        cnt = jnp.sum(ge.astype(jnp.float32), axis=0, keepdims=True)
        pred = cnt >= k
        return (jnp.where(pred, mid, lo), jnp.where(pred, hi, mid))

    lo0 = jnp.full((1, G), _MAP_LO, jnp.int32)
    hi0 = jnp.full((1, G), _MAP_HI, jnp.int32)
    t, _ = lax.fori_loop(0, 32, bs, (lo0, hi0))

    eqg = ((m == t) & oh).astype(jnp.float32)
    gtg = ((m > t) & oh).astype(jnp.float32)
    gt_node = jnp.sum(gtg, axis=1, keepdims=True)
    n_gt = jnp.sum(gtg, axis=0, keepdims=True)
    slack = k - n_gt
    ii = lax.broadcasted_iota(jnp.int32, (NP, 1), 0)

    def ts(i, carry):
        lo, hi = carry
        mid = lo + lax.shift_right_logical(hi - lo, 1)
        cl = jnp.sum(jnp.where(ii < mid, eqg, 0.0), axis=0, keepdims=True)
        pred = cl >= slack
        return (jnp.where(pred, lo, mid), jnp.where(pred, mid, hi))

    _, c = lax.fori_loop(0, 14, ts,
                         (jnp.zeros((1, G), jnp.int32),
                          jnp.full((1, G), NP, jnp.int32)))
    kept_eq = jnp.sum(jnp.where(ii < c, eqg, 0.0), axis=1, keepdims=True)
    kept = (gt_node > 0.0) | (kept_eq > 0.0)
    keepn_ref[...] = jnp.where(kept & keepb, 1.0, 0.0)


def _tck_select(score, keep, batch2):
    return pl.pallas_call(
        _select_body,
        out_shape=jax.ShapeDtypeStruct((NP, 1), jnp.float32),
    )(score, keep, batch2)


def _pool_body(has_next, h_ref, score_ref, keepn_ref, batch_ref, *rest):
    if has_next:
        wn_ref, hnew_ref, xcat_ref, xwn_ref = rest
    else:
        hnew_ref, xcat_ref = rest
    newkeep = keepn_ref[...]
    hnew = newkeep * score_ref[...] * h_ref[...]
    hnew_ref[...] = hnew
    batch2 = batch_ref[...]
    gids = lax.broadcasted_iota(jnp.int32, (1, G), 1)
    ohf = (batch2 == gids).astype(jnp.float32)
    dims = (((0,), (0,)), ((), ()))
    s = lax.dot_general(ohf, hnew, dims, preferred_element_type=jnp.float32)
    cntT = lax.dot_general(ohf, newkeep, dims, preferred_element_type=jnp.float32)
    gmean = s / jnp.maximum(cntT, 1.0)
    gids_col = lax.broadcasted_iota(jnp.int32, (G, 1), 0)

    def gm(g, out):
        colm = (batch2 == g) & (newkeep > 0.0)
        xm = jnp.where(colm, hnew, jnp.float32(-1e30))
        mg = jnp.max(xm, axis=0, keepdims=True)
        sel = (gids_col == g).astype(jnp.float32)
        return out + sel * mg

    gmax = lax.fori_loop(0, G, gm, jnp.zeros((G, H), jnp.float32))
    gmax = jnp.where(cntT > 0.0, gmax, 0.0)
    xcat_ref[...] = jnp.concatenate([gmax, gmean], axis=1)
    if has_next:
        xwn_ref[...] = jnp.dot(hnew, wn_ref[...],
                               preferred_element_type=jnp.float32)


def _tck_pool(h, score, keepn, batch2, wn):
    has_next = wn is not None
    out_shape = [
        jax.ShapeDtypeStruct((NP, H), jnp.float32),
        jax.ShapeDtypeStruct((G, 2 * H), jnp.float32),
    ]
    args = [h, score, keepn, batch2]
    if has_next:
        out_shape.append(jax.ShapeDtypeStruct((NP, H), jnp.float32))
        args.append(wn)
    body = functools.partial(_pool_body, has_next)
    return pl.pallas_call(body, out_shape=tuple(out_shape))(*args)


def _tck_layer(accp, deg, xw, keep, batch2, pw, b, wn):
    h, score = _tck_convscore(accp, deg, keep, xw, pw, b)
    keepn = _tck_select(score, keep, batch2)
    outs = _tck_pool(h, score, keepn, batch2, wn)
    if wn is None:
        hnew, xcat = outs
        return hnew, keepn, xcat
    hnew, xcat, xwn = outs
    return hnew, keepn, xcat, xwn


def _fin_body(accp_ref, deg_ref, h3_ref, keep_ref, xc1_ref, xc2_ref, xc3_ref,
              l1w_ref, l1b_ref, l2w_ref, l2b_ref, l3w_ref, l3b_ref,
              logits_ref, sm_ref):
    acc = accp_ref[0] + accp_ref[1]
    deg = deg_ref[...]
    dis = _safe_rsqrt(deg)
    keep = keep_ref[...]
    h3 = h3_ref[...]
    info = -(dis * acc) + h3 * keep
    ns = jnp.sum(jnp.abs(info), axis=1, keepdims=True) * keep
    sm = jnp.sum(ns) / jnp.maximum(jnp.sum(keep), 1.0)
    sm_ref[...] = jnp.full((1, 1), sm, jnp.float32)

    g = _leaky(xc1_ref[...]) + _leaky(xc2_ref[...]) + _leaky(xc3_ref[...])
    g = _leaky(jnp.dot(g, l1w_ref[...], preferred_element_type=jnp.float32)
               + l1b_ref[...])
    g = _leaky(jnp.dot(g, l2w_ref[...], preferred_element_type=jnp.float32)
               + l2b_ref[...])
    logits_ref[...] = (jnp.dot(g, l3w_ref[...], preferred_element_type=jnp.float32)
                       + l3b_ref[...])


def _tck_fin(accp, degp, h3, keep, xc1, xc2, xc3, l1w, l1b, l2w, l2b, l3w, l3b):
    return pl.pallas_call(
        _fin_body,
        out_shape=(
            jax.ShapeDtypeStruct((G, 128), jnp.float32),
            jax.ShapeDtypeStruct((1, 1), jnp.float32),
        ),
    )(accp, degp, h3, keep, xc1, xc2, xc3, l1w, l1b, l2w, l2b, l3w, l3b)


def kernel(x, edge_index, batch, W1, b1, W2, b2, W3, b3, pw1, pw2, pw3,
           L1W, L1b, L2W, L2b, L3W, L3b):
    f32 = jnp.float32
    src = edge_index[0].astype(jnp.int32)
    dst = edge_index[1].astype(jnp.int32)
    src_r = jnp.concatenate(
        [src, jnp.zeros((EP - E,), jnp.int32)]).reshape(NW, NBLK, 128)
    dstp0 = jnp.concatenate(
        [dst, jnp.full((EP - E,), DUMMY, jnp.int32)]).reshape(NW, NBLK, 128)
    batch2 = jnp.concatenate(
        [batch.astype(jnp.int32), jnp.full((NP - N,), G, jnp.int32)]
    ).reshape(NP, 1)
    keep1_flat = jnp.concatenate(
        [jnp.ones((N,), f32), jnp.zeros((NP - N,), f32)])
    keep1 = keep1_flat.reshape(NP, 1)
    xp = jnp.pad(x, ((0, NP - N), (0, 0)))
    b1r = b1.reshape(1, H)
    b2r = b2.reshape(1, H)
    b3r = b3.reshape(1, H)
    pw1r = pw1.reshape(H, 1)
    pw2r = pw2.reshape(H, 1)
    pw3r = pw3.reshape(H, 1)
    l2wp = jnp.pad(L2W, ((0, 0), (0, 128 - H // 2)))
    l2bp = jnp.pad(L2b, (0, 128 - H // 2)).reshape(1, 128)
    l3wp = jnp.pad(L3W, ((0, 128 - H // 2), (0, 128 - C)))
    l3bp = jnp.pad(L3b, (0, 128 - C)).reshape(1, 128)

    dstp1, degp1 = _edge_prep_conv(keep1_flat, src_r, dstp0)
    deg1 = _tck_degsum(degp1).reshape(NP, 1)
    xw1, hh1 = _tck_first(xp, W1, deg1, keep1)
    accp1 = _propagate(hh1, src_r, dstp1)
    h1, keep2, xc1, xw2 = _tck_layer(accp1, deg1, xw1, keep1, batch2,
                                     pw1r, b1r, W2)

    dstp2, degp2 = _edge_prep_conv(keep2.reshape(NP), src_r, dstp1)
    deg2 = _tck_degsum(degp2).reshape(NP, 1)
    hh2 = _tck_scale(xw2, deg2, keep2)
    accp2 = _propagate(hh2, src_r, dstp2)
    h2, keep3, xc2, xw3 = _tck_layer(accp2, deg2, xw2, keep2, batch2,
                                     pw2r, b2r, W3)

    dstp3, degp3 = _edge_prep_conv(keep3.reshape(NP), src_r, dstp2)
    deg3 = _tck_degsum(degp3).reshape(NP, 1)
    hh3 = _tck_scale(xw3, deg3, keep3)
    accp3 = _propagate(hh3, src_r, dstp3)
    h3, keep4, xc3 = _tck_layer(accp3, deg3, xw3, keep3, batch2,
                                pw3r, b3r, None)

    dstpn, degpn = _edge_prep_ni(src_r, dstp3)
    degn = _tck_degsum(degpn).reshape(NP, 1)
    hhn = _tck_scale_ni(h3, degn)
    accpn = _propagate(hhn, src_r, dstpn)
    logits_p, smp = _tck_fin(accpn, degn, h3, keep4, xc1, xc2, xc3,
                             L1W, L1b.reshape(1, H), l2wp, l2bp, l3wp, l3bp)
    return logits_p[:, :C], smp.reshape(())
